# baseline (speedup 1.0000x reference)
import functools

import jax
import jax.numpy as jnp
import numpy as np
from jax.experimental import pallas as pl
from jax.experimental.pallas import tpu as pltpu

B, S, H = 2, 8192, 2048
NH = 16
D = H // NH
CHUNK = 2048
NC = S // CHUNK
BH = B * NH
EPS = 1e-6
BASE_LR_INV = float(np.log(np.expm1(1e-3)))
ROPE_BASE = 1000000.0

BM = 512
MT = (B * S) // BM


def _silu(x):
    return x * jax.nn.sigmoid(x)


def _qkv_kernel(x_ref, w_ref, aw_ref, bw_ref, lw_ref, lb_ref,
                act_ref, lrm_ref):
    j = pl.program_id(0)
    x = x_ref[...]
    acc = jax.lax.dot_general(
        x, w_ref[...], (((1,), (1,)), ((), ())),
        preferred_element_type=jnp.float32)

    @pl.when(j < 2)
    def _():
        v = jnp.mean(acc * acc, axis=-1, keepdims=True)
        y = acc * jax.lax.rsqrt(v + EPS) * aw_ref[0] + bw_ref[0]
        z = _silu(y)
        for g in range(NH):
            sl = slice(g * D, (g + 1) * D)
            zz = z[:, sl]
            nrm = jax.lax.rsqrt(
                jnp.sum(zz * zz, axis=-1, keepdims=True) + 1e-6)
            act_ref[0, :, sl] = (zz * nrm).astype(jnp.bfloat16)

    @pl.when(j == 2)
    def _():
        act_ref[0] = _silu(acc).astype(jnp.bfloat16)

    @pl.when(j == 2)
    def _():
        z = jax.lax.dot_general(
            x, lw_ref[...], (((1,), (1,)), ((), ())),
            preferred_element_type=jnp.float32) + lb_ref[...]
        lane = jax.lax.broadcasted_iota(jnp.int32, z.shape, 1)
        lrm_ref[...] = jnp.where(lane < 3 * NH,
                                 jax.nn.softplus(z), jax.nn.sigmoid(z))


def _run_qkv(x2d, qkv_w, aw, bw, lw, lb):
    grid = (3, MT)
    act, lrm = pl.pallas_call(
        _qkv_kernel,
        grid=grid,
        in_specs=[
            pl.BlockSpec((BM, H), lambda j, m: (m, 0)),
            pl.BlockSpec((H, H), lambda j, m: (j, 0)),
            pl.BlockSpec((1, 1, H), lambda j, m: (j, 0, 0)),
            pl.BlockSpec((1, 1, H), lambda j, m: (j, 0, 0)),
            pl.BlockSpec((4 * NH, H), lambda j, m: (0, 0)),
            pl.BlockSpec((1, 4 * NH), lambda j, m: (0, 0)),
        ],
        out_specs=[
            pl.BlockSpec((1, BM, H), lambda j, m: (j, m, 0)),
            pl.BlockSpec((BM, 4 * NH), lambda j, m: (m, 0)),
        ],
        out_shape=[
            jax.ShapeDtypeStruct((3, B * S, H), jnp.bfloat16),
            jax.ShapeDtypeStruct((B * S, 4 * NH), jnp.float32),
        ],
        compiler_params=pltpu.CompilerParams(
            dimension_semantics=("arbitrary", "arbitrary"),
            vmem_limit_bytes=56 * 1024 * 1024,
        ),
        name="qkv_fused",
    )(x2d, qkv_w, aw, bw, lw, lb)
    return act, lrm


def _scan_kernel(qkv_ref, cos_ref, sin_ref, lrm_ref,
                 w02_ref, w1_ref, ttw_ref, out_ref,
                 w02s, w1s, m02s, m1s):
    c = pl.program_id(1)

    @pl.when(c == 0)
    def _():
        w02s[...] = w02_ref[0]
        w1s[...] = w1_ref[0]
        m02s[...] = jnp.zeros_like(m02s)
        m1s[...] = jnp.zeros_like(m1s)

    c64 = cos_ref[0, 0]
    s64 = sin_ref[0, 0]
    cos = jnp.concatenate([c64, c64], axis=-1)
    sin = jnp.concatenate([-s64, s64], axis=-1)

    ji = jax.lax.broadcasted_iota(jnp.int32, (D, D), 0)
    di = jax.lax.broadcasted_iota(jnp.int32, (D, D), 1)
    rmat = jnp.where(di == ((ji + D // 2) % D), 1.0, 0.0).astype(jnp.float32)

    def prep(x):
        xr = jnp.dot(x, rmat, preferred_element_type=jnp.float32)
        return x * cos + xr * sin

    q = prep(qkv_ref[0, 0, 0].astype(jnp.float32))
    k = prep(qkv_ref[1, 0, 0].astype(jnp.float32))
    v = qkv_ref[2, 0, 0].astype(jnp.float32)

    w02 = w02s[...]
    w1 = w1s[...]

    qh = jnp.dot(q, w02, preferred_element_type=jnp.float32)
    hid_q = _silu(qh[:, :D]) * qh[:, D:]
    o = jax.lax.dot_general(hid_q, w1, (((1,), (1,)), ((), ())),
                            preferred_element_type=jnp.float32)

    kh = jnp.dot(k, w02, preferred_element_type=jnp.float32)
    g = kh[:, :D]
    u = kh[:, D:]
    sg = jax.nn.sigmoid(g)
    sig = g * sg
    hid_k = sig * u
    dh = jnp.dot(v, w1, preferred_element_type=jnp.float32)
    du = dh * sig
    dgb = dh * u * sg * (1.0 + g * (1.0 - sg))

    lrm = lrm_ref[0, 0]
    l1 = lrm[:, 0:1]
    l2 = lrm[:, 1:2]
    l3 = lrm[:, 2:3]
    mbar = jnp.mean(lrm[:, 3:4], axis=0, keepdims=True)

    a = jnp.concatenate([dgb * l1, du * l3], axis=-1)
    dw02 = jax.lax.dot_general(k, a, (((0,), (0,)), ((), ())),
                               preferred_element_type=jnp.float32)
    dw1 = jax.lax.dot_general(v * l2, hid_k, (((0,), (0,)), ((), ())),
                              preferred_element_type=jnp.float32)

    m02 = mbar * m02s[...] + dw02
    m1 = mbar * m1s[...] + dw1
    m02s[...] = m02
    m1s[...] = m1

    w02n = w02 + m02
    w02s[...] = w02n * jax.lax.rsqrt(
        jnp.sum(w02n * w02n, axis=0, keepdims=True) + 1e-6)
    w1n = w1 + m1
    w1s[...] = w1n * jax.lax.rsqrt(
        jnp.sum(w1n * w1n, axis=-1, keepdims=True) + 1e-6)

    mo = jnp.mean(o * o, axis=-1, keepdims=True)
    out_ref[0] = ttw_ref[...] * (o * jax.lax.rsqrt(mo + EPS))


def _run_scan(act, cos4, sin4, lrm4, w02T, w1_all, ttw):
    grid = (BH, NC)
    act5 = act.reshape(3, BH, NC, CHUNK, D)
    out = pl.pallas_call(
        _scan_kernel,
        grid=grid,
        in_specs=[
            pl.BlockSpec((3, 1, 1, CHUNK, D), lambda g, c: (0, g, c, 0, 0)),
            pl.BlockSpec((1, 1, CHUNK, D // 2), lambda g, c: (g // NH, c, 0, 0)),
            pl.BlockSpec((1, 1, CHUNK, D // 2), lambda g, c: (g // NH, c, 0, 0)),
            pl.BlockSpec((1, 1, CHUNK, 4), lambda g, c: (g, c, 0, 0)),
            pl.BlockSpec((1, D, 2 * D), lambda g, c: (g % NH, 0, 0)),
            pl.BlockSpec((1, D, D), lambda g, c: (g % NH, 0, 0)),
            pl.BlockSpec((1, D), lambda g, c: (0, 0)),
        ],
        out_specs=pl.BlockSpec((1, CHUNK, D),
                               lambda g, c: (g // NH, c, g % NH)),
        out_shape=jax.ShapeDtypeStruct((B, S, H), jnp.float32),
        scratch_shapes=[
            pltpu.VMEM((D, 2 * D), jnp.float32),
            pltpu.VMEM((D, D), jnp.float32),
            pltpu.VMEM((D, 2 * D), jnp.float32),
            pltpu.VMEM((D, D), jnp.float32),
        ],
        compiler_params=pltpu.CompilerParams(
            dimension_semantics=("parallel", "arbitrary"),
            vmem_limit_bytes=48 * 1024 * 1024,
        ),
        name="lact_scan",
    )(act5, cos4, sin4, lrm4, w02T, w1_all, ttw)
    return out


@jax.jit
def kernel(hidden_states, position_ids, qkv_w, q_norm_w, k_norm_w, qk_scale,
           qk_offset, w0, w1, w2, lr_w, lr_b, mom_w, mom_b, ttt_norm_w):
    x2d = hidden_states.reshape(B * S, H)

    aw = jnp.stack([q_norm_w * qk_scale[:, 0],
                    k_norm_w * qk_scale[:, 1],
                    jnp.ones((H,), jnp.float32)], axis=0).reshape(3, 1, H)
    bw = jnp.stack([qk_offset[:, 0], qk_offset[:, 1],
                    jnp.zeros((H,), jnp.float32)], axis=0).reshape(3, 1, H)
    lw = jnp.concatenate([lr_w, mom_w], axis=0)
    lb = jnp.concatenate([lr_b + BASE_LR_INV, mom_b])[None, :]

    act, lrm = _run_qkv(x2d.astype(jnp.bfloat16), qkv_w.astype(jnp.bfloat16),
                        aw, bw, lw.astype(jnp.bfloat16), lb)

    inv_freq = 1.0 / (ROPE_BASE ** (jnp.arange(0, D, 2, dtype=jnp.float32) / D))
    freqs = position_ids[:, :, None].astype(jnp.float32) * inv_freq[None, None, :]
    cos4 = jnp.cos(freqs).reshape(B, NC, CHUNK, D // 2)
    sin4 = jnp.sin(freqs).reshape(B, NC, CHUNK, D // 2)

    lrm4 = jnp.concatenate(
        [lrm[:, :3 * NH].reshape(B, S, NH, 3),
         lrm[:, 3 * NH:].reshape(B, S, NH, 1)], axis=-1)
    lrm4 = lrm4.transpose(0, 2, 1, 3).reshape(BH, NC, CHUNK, 4)

    w02T = jnp.concatenate([w0.transpose(0, 2, 1),
                            w2.transpose(0, 2, 1)], axis=-1)
    ttw = ttt_norm_w[None, :]

    return _run_scan(act, cos4, sin4, lrm4, w02T, w1, ttw)


# speedup vs baseline: 1.0266x; 1.0266x over previous
import functools

import jax
import jax.numpy as jnp
import numpy as np
from jax.experimental import pallas as pl
from jax.experimental.pallas import tpu as pltpu

B, S, H = 2, 8192, 2048
NH = 16
D = H // NH
CHUNK = 2048
NC = S // CHUNK
BH = B * NH
EPS = 1e-6
BASE_LR_INV = float(np.log(np.expm1(1e-3)))
ROPE_BASE = 1000000.0

BM = 512
MT = (B * S) // BM


def _silu(x):
    return x * jax.nn.sigmoid(x)


def _qkv_kernel(x_ref, w_ref, aw_ref, bw_ref, lw_ref, lb_ref,
                act_ref, lrm_ref):
    j = pl.program_id(0)
    x = x_ref[...]
    acc = jax.lax.dot_general(
        x, w_ref[...], (((1,), (1,)), ((), ())),
        preferred_element_type=jnp.float32)

    @pl.when(j < 2)
    def _():
        v = jnp.mean(acc * acc, axis=-1, keepdims=True)
        y = acc * jax.lax.rsqrt(v + EPS) * aw_ref[0] + bw_ref[0]
        z = _silu(y)
        for g in range(NH):
            sl = slice(g * D, (g + 1) * D)
            zz = z[:, sl]
            nrm = jax.lax.rsqrt(
                jnp.sum(zz * zz, axis=-1, keepdims=True) + 1e-6)
            act_ref[0, :, sl] = (zz * nrm).astype(jnp.bfloat16)

    @pl.when(j == 2)
    def _():
        act_ref[0] = _silu(acc).astype(jnp.bfloat16)

    @pl.when(j == 2)
    def _():
        z = jax.lax.dot_general(
            x, lw_ref[...], (((1,), (1,)), ((), ())),
            preferred_element_type=jnp.float32) + lb_ref[...]
        lane = jax.lax.broadcasted_iota(jnp.int32, z.shape, 1)
        lrm_ref[...] = jnp.where(lane < 3 * NH,
                                 jax.nn.softplus(z), jax.nn.sigmoid(z))


def _run_qkv(x2d, qkv_w, aw, bw, lw, lb):
    grid = (3, MT)
    act, lrm = pl.pallas_call(
        _qkv_kernel,
        grid=grid,
        in_specs=[
            pl.BlockSpec((BM, H), lambda j, m: (m, 0)),
            pl.BlockSpec((H, H), lambda j, m: (j, 0)),
            pl.BlockSpec((1, 1, H), lambda j, m: (j, 0, 0)),
            pl.BlockSpec((1, 1, H), lambda j, m: (j, 0, 0)),
            pl.BlockSpec((4 * NH, H), lambda j, m: (0, 0)),
            pl.BlockSpec((1, 4 * NH), lambda j, m: (0, 0)),
        ],
        out_specs=[
            pl.BlockSpec((1, BM, H), lambda j, m: (j, m, 0)),
            pl.BlockSpec((BM, 4 * NH), lambda j, m: (m, 0)),
        ],
        out_shape=[
            jax.ShapeDtypeStruct((3, B * S, H), jnp.bfloat16),
            jax.ShapeDtypeStruct((B * S, 4 * NH), jnp.float32),
        ],
        compiler_params=pltpu.CompilerParams(
            dimension_semantics=("arbitrary", "arbitrary"),
            vmem_limit_bytes=56 * 1024 * 1024,
        ),
        name="qkv_fused",
    )(x2d, qkv_w, aw, bw, lw, lb)
    return act, lrm


def _scan_kernel(qkv_ref, cos_ref, sin_ref, lrm_ref,
                 w02_ref, w1_ref, ttw_ref, out_ref,
                 w02s, w1s, m02s, m1s):
    c = pl.program_id(1)

    @pl.when(c == 0)
    def _():
        w02s[...] = w02_ref[...]
        w1s[...] = w1_ref[...]
        m02s[...] = jnp.zeros_like(m02s)
        m1s[...] = jnp.zeros_like(m1s)

    c64 = cos_ref[0, 0]
    s64 = sin_ref[0, 0]
    cos = jnp.concatenate([c64, c64], axis=-1)
    sin = jnp.concatenate([-s64, s64], axis=-1)

    ji = jax.lax.broadcasted_iota(jnp.int32, (D, D), 0)
    di = jax.lax.broadcasted_iota(jnp.int32, (D, D), 1)
    rmat = jnp.where(di == ((ji + D // 2) % D), 1.0, 0.0).astype(jnp.float32)

    def prep(x):
        xr = jnp.dot(x, rmat, preferred_element_type=jnp.float32)
        return x * cos + xr * sin

    for s in range(2):
        q = prep(qkv_ref[0, s, 0].astype(jnp.float32))
        k = prep(qkv_ref[1, s, 0].astype(jnp.float32))
        v = qkv_ref[2, s, 0].astype(jnp.float32)

        w02 = w02s[s]
        w1 = w1s[s]

        qh = jnp.dot(q, w02, preferred_element_type=jnp.float32)
        hid_q = _silu(qh[:, :D]) * qh[:, D:]
        o = jax.lax.dot_general(hid_q, w1, (((1,), (1,)), ((), ())),
                                preferred_element_type=jnp.float32)

        kh = jnp.dot(k, w02, preferred_element_type=jnp.float32)
        g = kh[:, :D]
        u = kh[:, D:]
        sg = jax.nn.sigmoid(g)
        sig = g * sg
        hid_k = sig * u
        dh = jnp.dot(v, w1, preferred_element_type=jnp.float32)
        du = dh * sig
        dgb = dh * u * sg * (1.0 + g * (1.0 - sg))

        lrm = lrm_ref[s, 0]
        l1 = lrm[:, 0:1]
        l2 = lrm[:, 1:2]
        l3 = lrm[:, 2:3]
        mbar = jnp.mean(lrm[:, 3:4], axis=0, keepdims=True)

        a = jnp.concatenate([dgb * l1, du * l3], axis=-1)
        dw02 = jax.lax.dot_general(k, a, (((0,), (0,)), ((), ())),
                                   preferred_element_type=jnp.float32)
        dw1 = jax.lax.dot_general(v * l2, hid_k, (((0,), (0,)), ((), ())),
                                  preferred_element_type=jnp.float32)

        m02 = mbar * m02s[s] + dw02
        m1 = mbar * m1s[s] + dw1
        m02s[s] = m02
        m1s[s] = m1

        w02n = w02 + m02
        w02s[s] = w02n * jax.lax.rsqrt(
            jnp.sum(w02n * w02n, axis=0, keepdims=True) + 1e-6)
        w1n = w1 + m1
        w1s[s] = w1n * jax.lax.rsqrt(
            jnp.sum(w1n * w1n, axis=-1, keepdims=True) + 1e-6)

        mo = jnp.mean(o * o, axis=-1, keepdims=True)
        out_ref[0, :, s * D:(s + 1) * D] = ttw_ref[...] * (
            o * jax.lax.rsqrt(mo + EPS))


def _run_scan(act, cos4, sin4, lrm4, w02T, w1_all, ttw):
    grid = (BH // 2, NC)
    act5 = act.reshape(3, BH, NC, CHUNK, D)
    out = pl.pallas_call(
        _scan_kernel,
        grid=grid,
        in_specs=[
            pl.BlockSpec((3, 2, 1, CHUNK, D), lambda g, c: (0, g, c, 0, 0)),
            pl.BlockSpec((1, 1, CHUNK, D // 2), lambda g, c: (g // 8, c, 0, 0)),
            pl.BlockSpec((1, 1, CHUNK, D // 2), lambda g, c: (g // 8, c, 0, 0)),
            pl.BlockSpec((2, 1, CHUNK, 4), lambda g, c: (g, c, 0, 0)),
            pl.BlockSpec((2, D, 2 * D), lambda g, c: (g % 8, 0, 0)),
            pl.BlockSpec((2, D, D), lambda g, c: (g % 8, 0, 0)),
            pl.BlockSpec((1, D), lambda g, c: (0, 0)),
        ],
        out_specs=pl.BlockSpec((1, CHUNK, 2 * D),
                               lambda g, c: (g // 8, c, g % 8)),
        out_shape=jax.ShapeDtypeStruct((B, S, H), jnp.float32),
        scratch_shapes=[
            pltpu.VMEM((2, D, 2 * D), jnp.float32),
            pltpu.VMEM((2, D, D), jnp.float32),
            pltpu.VMEM((2, D, 2 * D), jnp.float32),
            pltpu.VMEM((2, D, D), jnp.float32),
        ],
        compiler_params=pltpu.CompilerParams(
            dimension_semantics=("parallel", "arbitrary"),
            vmem_limit_bytes=48 * 1024 * 1024,
        ),
        name="lact_scan",
    )(act5, cos4, sin4, lrm4, w02T, w1_all, ttw)
    return out


@jax.jit
def kernel(hidden_states, position_ids, qkv_w, q_norm_w, k_norm_w, qk_scale,
           qk_offset, w0, w1, w2, lr_w, lr_b, mom_w, mom_b, ttt_norm_w):
    x2d = hidden_states.reshape(B * S, H)

    aw = jnp.stack([q_norm_w * qk_scale[:, 0],
                    k_norm_w * qk_scale[:, 1],
                    jnp.ones((H,), jnp.float32)], axis=0).reshape(3, 1, H)
    bw = jnp.stack([qk_offset[:, 0], qk_offset[:, 1],
                    jnp.zeros((H,), jnp.float32)], axis=0).reshape(3, 1, H)
    lw = jnp.concatenate([lr_w, mom_w], axis=0)
    lb = jnp.concatenate([lr_b + BASE_LR_INV, mom_b])[None, :]

    act, lrm = _run_qkv(x2d.astype(jnp.bfloat16), qkv_w.astype(jnp.bfloat16),
                        aw, bw, lw.astype(jnp.bfloat16), lb)

    inv_freq = 1.0 / (ROPE_BASE ** (jnp.arange(0, D, 2, dtype=jnp.float32) / D))
    freqs = position_ids[:, :, None].astype(jnp.float32) * inv_freq[None, None, :]
    cos4 = jnp.cos(freqs).reshape(B, NC, CHUNK, D // 2)
    sin4 = jnp.sin(freqs).reshape(B, NC, CHUNK, D // 2)

    lrm4 = jnp.concatenate(
        [lrm[:, :3 * NH].reshape(B, S, NH, 3),
         lrm[:, 3 * NH:].reshape(B, S, NH, 1)], axis=-1)
    lrm4 = lrm4.transpose(0, 2, 1, 3).reshape(BH, NC, CHUNK, 4)

    w02T = jnp.concatenate([w0.transpose(0, 2, 1),
                            w2.transpose(0, 2, 1)], axis=-1)
    ttw = ttt_norm_w[None, :]

    return _run_scan(act, cos4, sin4, lrm4, w02T, w1, ttw)
